# baseline (speedup 1.0000x reference)
import functools

import jax
import jax.numpy as jnp
import numpy as np
from jax import lax
from jax.experimental import pallas as pl
from jax.experimental.pallas import tpu as pltpu
from jax.experimental.pallas import tpu_sc as plsc

N = 10000
NPAD = 10240
E = 320000
CHUNK = 128
NCHUNKS = 2560
CH_PER_TILE = NCHUNKS // 32
EPAD = NCHUNKS * CHUNK
HEADS = 8
BASES = 4
F_H = 16
F_B = BASES * F_H
ROWS_PER_TILE = NPAD // 16
BLK = 256
GRID = NPAD // BLK

_mesh = plsc.VectorSubcoreMesh(core_axis_name="c", subcore_axis_name="s")
_sc_params = pltpu.CompilerParams(use_tc_tiling_on_sc=False)
_sc_params_nl = pltpu.CompilerParams(use_tc_tiling_on_sc=False,
                                     needs_layout_passes=False)


@functools.partial(
    pl.kernel,
    out_type=jax.ShapeDtypeStruct((2, NPAD, 16), jnp.float32),
    mesh=_mesh,
    compiler_params=_sc_params,
    scratch_types=[
        pltpu.VMEM((CH_PER_TILE, CHUNK), jnp.int32),
        pltpu.VMEM((CHUNK, 16), jnp.float32),
        pltpu.VMEM_SHARED((NPAD, 16), jnp.float32),
        pltpu.SemaphoreType.DMA,
    ],
)
def _sc_degree(col_hbm, out_hbm, col_v, ones_v, deg_sh, sem):
    cid = lax.axis_index("c")
    sid = lax.axis_index("s")
    wid = sid * 2 + cid

    z16 = jnp.zeros((16,), jnp.float32)

    @pl.loop(0, CHUNK)
    def _(i):
        ones_v[i, pl.ds(0, 16)] = z16

    @pl.loop(0, ROWS_PER_TILE // CHUNK)
    def _(k):
        pltpu.sync_copy(ones_v, deg_sh.at[pl.ds(sid * ROWS_PER_TILE + k * CHUNK, CHUNK)])

    o16 = jnp.ones((16,), jnp.float32)

    @pl.loop(0, CHUNK)
    def _(i):
        ones_v[i, pl.ds(0, 16)] = o16

    pltpu.sync_copy(col_hbm.at[pl.ds(wid * CH_PER_TILE, CH_PER_TILE)], col_v)
    plsc.subcore_barrier()

    @pl.loop(0, CH_PER_TILE)
    def _(j):
        pltpu.async_copy(ones_v, deg_sh.at[col_v.at[j]], sem, add=True)

    @pl.loop(0, CH_PER_TILE)
    def _(j):
        pltpu.make_async_copy(ones_v, deg_sh.at[col_v.at[j]], sem).wait()

    plsc.subcore_barrier()
    pltpu.sync_copy(
        deg_sh.at[pl.ds(sid * ROWS_PER_TILE, ROWS_PER_TILE)],
        out_hbm.at[cid, pl.ds(sid * ROWS_PER_TILE, ROWS_PER_TILE)],
    )


ECHUNK = 2048
F_T = F_B // 16
ECORE = EPAD // 2
NECH = ECORE // ECHUNK


@functools.partial(
    pl.kernel,
    out_type=jax.ShapeDtypeStruct((2, 16, NPAD * F_T), jnp.float32),
    mesh=_mesh,
    compiler_params=_sc_params_nl,
    scratch_types=[
        pltpu.VMEM((NPAD * F_T,), jnp.float32),
        pltpu.VMEM((NPAD * F_T,), jnp.float32),
        pltpu.VMEM((2, ECHUNK), jnp.int32),
        pltpu.VMEM((2, ECHUNK), jnp.int32),
        pltpu.SemaphoreType.DMA,
        pltpu.SemaphoreType.DMA,
        pltpu.SemaphoreType.DMA,
    ],
)
def _sc_agg(b2_hbm, row_hbm, col_hbm, out_hbm, b2_v, agg_v, row_v, col_v,
            sem_r, sem_c, sem_b):
    cid = lax.axis_index("c")
    sid = lax.axis_index("s")

    b2cp = pltpu.async_copy(b2_hbm.at[sid], b2_v, sem_b)

    z16 = jnp.zeros((16,), jnp.float32)

    @pl.loop(0, NPAD * F_T, step=16)
    def _(i):
        agg_v[pl.ds(i, 16)] = z16

    ebase = cid * ECORE

    def _fetch(ch, slot):
        pltpu.async_copy(
            row_hbm.at[pl.ds(ebase + ch * ECHUNK, ECHUNK)], row_v.at[slot], sem_r)
        pltpu.async_copy(
            col_hbm.at[pl.ds(ebase + ch * ECHUNK, ECHUNK)], col_v.at[slot], sem_c)

    def _wait(ch, slot):
        pltpu.make_async_copy(
            row_hbm.at[pl.ds(ebase + ch * ECHUNK, ECHUNK)], row_v.at[slot], sem_r).wait()
        pltpu.make_async_copy(
            col_hbm.at[pl.ds(ebase + ch * ECHUNK, ECHUNK)], col_v.at[slot], sem_c).wait()

    def _process(slot):
        @plsc.parallel_loop(0, ECHUNK, step=16, unroll=8)
        def _(e):
            r4 = row_v[slot, pl.ds(e, 16)] << 2
            c4 = col_v[slot, pl.ds(e, 16)] << 2
            for f in range(F_T):
                rf = r4 if f == 0 else r4 + f
                cf = c4 if f == 0 else c4 + f
                v = plsc.load_gather(b2_v, [rf])
                plsc.addupdate_scatter(agg_v, [cf], v)

    _fetch(0, 0)
    b2cp.wait()

    @pl.loop(0, NECH, step=2)
    def _(ch):
        _wait(ch, 0)
        _fetch(ch + 1, 1)
        _process(0)
        _wait(ch + 1, 1)

        @pl.when(ch + 2 < NECH)
        def _():
            _fetch(ch + 2, 0)

        _process(1)

    pltpu.sync_copy(agg_v, out_hbm.at[cid, sid])


def _dense_body(x_ref, wb_ref, wc_ref, bc_ref, wr_ref, br_ref, b_ref, wt_ref, r_ref):
    xb = x_ref[...]
    b_ref[...] = jnp.dot(xb, wb_ref[...], preferred_element_type=jnp.float32)
    wt_ref[...] = jnp.dot(xb, wc_ref[...], preferred_element_type=jnp.float32) + bc_ref[...]
    r_ref[...] = jnp.dot(xb, wr_ref[...], preferred_element_type=jnp.float32) + br_ref[...]


_dense = pl.pallas_call(
    _dense_body,
    grid=(GRID,),
    in_specs=[
        pl.BlockSpec((BLK, 128), lambda i: (i, 0)),
        pl.BlockSpec((128, F_B), lambda i: (0, 0)),
        pl.BlockSpec((128, HEADS * BASES), lambda i: (0, 0)),
        pl.BlockSpec((1, HEADS * BASES), lambda i: (0, 0)),
        pl.BlockSpec((128, 128), lambda i: (0, 0)),
        pl.BlockSpec((1, 128), lambda i: (0, 0)),
    ],
    out_specs=[
        pl.BlockSpec((BLK, F_B), lambda i: (i, 0)),
        pl.BlockSpec((BLK, HEADS * BASES), lambda i: (i, 0)),
        pl.BlockSpec((BLK, 128), lambda i: (i, 0)),
    ],
    out_shape=[
        jax.ShapeDtypeStruct((NPAD, F_B), jnp.float32),
        jax.ShapeDtypeStruct((NPAD, HEADS * BASES), jnp.float32),
        jax.ShapeDtypeStruct((NPAD, 128), jnp.float32),
    ],
)


def _scale_body(d0_ref, d1_ref, bases_ref, b2_ref, dis_ref):
    deg = d0_ref[:, 0:1] + d1_ref[:, 0:1] + 1.0
    dis = lax.rsqrt(deg)
    dis_ref[...] = dis
    b2_ref[...] = bases_ref[...] * dis


_scale = pl.pallas_call(
    _scale_body,
    grid=(GRID,),
    in_specs=[
        pl.BlockSpec((BLK, 16), lambda i: (i, 0)),
        pl.BlockSpec((BLK, 16), lambda i: (i, 0)),
        pl.BlockSpec((BLK, F_B), lambda i: (i, 0)),
    ],
    out_specs=[
        pl.BlockSpec((BLK, F_B), lambda i: (i, 0)),
        pl.BlockSpec((BLK, 1), lambda i: (i, 0)),
    ],
    out_shape=[
        jax.ShapeDtypeStruct((NPAD, F_B), jnp.float32),
        jax.ShapeDtypeStruct((NPAD, 1), jnp.float32),
    ],
)


_P_np = np.zeros((BASES, HEADS * BASES, 128), np.float32)
_Q_np = np.zeros((BASES, F_B, 128), np.float32)
for _b in range(BASES):
    for _h in range(HEADS):
        for _f in range(F_H):
            _P_np[_b, _h * BASES + _b, _h * F_H + _f] = 1.0
            _Q_np[_b, _b * F_H + _f, _h * F_H + _f] = 1.0


def _finish_body(a0_ref, a1_ref, dis_ref, bases_ref, wt_ref, res_ref, bc_ref,
                 g_ref, bt_ref, p_ref, q_ref, o_ref):
    dis = dis_ref[...]
    aggf = dis * (a0_ref[...] + a1_ref[...]) + (dis * dis) * bases_ref[...]
    wt = wt_ref[...]
    conv = None
    for b in range(BASES):
        we = jnp.dot(wt, p_ref[b], preferred_element_type=jnp.float32)
        ae = jnp.dot(aggf, q_ref[b], preferred_element_type=jnp.float32)
        t = we * ae
        conv = t if conv is None else conv + t
    o = conv + bc_ref[...] + res_ref[...]
    mu = jnp.mean(o, axis=1, keepdims=True)
    var = jnp.mean((o - mu) * (o - mu), axis=1, keepdims=True)
    o = (o - mu) * lax.rsqrt(var + 1e-5) * g_ref[...] + bt_ref[...]
    o_ref[...] = jnp.maximum(o, 0.0)


_finish = pl.pallas_call(
    _finish_body,
    grid=(GRID,),
    in_specs=[
        pl.BlockSpec((BLK, F_B), lambda i: (i, 0)),
        pl.BlockSpec((BLK, F_B), lambda i: (i, 0)),
        pl.BlockSpec((BLK, 1), lambda i: (i, 0)),
        pl.BlockSpec((BLK, F_B), lambda i: (i, 0)),
        pl.BlockSpec((BLK, HEADS * BASES), lambda i: (i, 0)),
        pl.BlockSpec((BLK, 128), lambda i: (i, 0)),
        pl.BlockSpec((1, 128), lambda i: (0, 0)),
        pl.BlockSpec((1, 128), lambda i: (0, 0)),
        pl.BlockSpec((1, 128), lambda i: (0, 0)),
        pl.BlockSpec((BASES, HEADS * BASES, 128), lambda i: (0, 0, 0)),
        pl.BlockSpec((BASES, F_B, 128), lambda i: (0, 0, 0)),
    ],
    out_specs=pl.BlockSpec((BLK, 128), lambda i: (i, 0)),
    out_shape=jax.ShapeDtypeStruct((NPAD, 128), jnp.float32),
)


def kernel(x, edge_index, W_bases, W_comb, b_comb, bias_conv, W_res, b_res,
           ln_gamma, ln_beta):
    x_pad = jnp.zeros((NPAD, 128), jnp.float32).at[:N].set(x)
    row = edge_index[0]
    col = edge_index[1]
    pad = jnp.full((EPAD - E,), N, jnp.int32)
    row_p = jnp.concatenate([row, pad]).reshape(NCHUNKS, CHUNK)
    col_p = jnp.concatenate([col, pad]).reshape(NCHUNKS, CHUNK)

    bases, wt, res = _dense(x_pad, W_bases, W_comb, b_comb.reshape(1, -1),
                            W_res, b_res.reshape(1, -1))
    degp = _sc_degree(col_p)
    b2, dis = _scale(degp[0], degp[1], bases)
    b2_blk = b2.reshape(NPAD, 16, F_T).transpose(1, 0, 2).reshape(16, NPAD * F_T)
    agg_t = _sc_agg(b2_blk, row_p.reshape(-1), col_p.reshape(-1))
    aggp = (agg_t.reshape(2, 16, NPAD, F_T).transpose(0, 2, 1, 3)
            .reshape(2, NPAD, F_B))
    out = _finish(aggp[0], aggp[1], dis, bases, wt, res,
                  bias_conv.reshape(1, -1), ln_gamma.reshape(1, -1),
                  ln_beta.reshape(1, -1), jnp.asarray(_P_np), jnp.asarray(_Q_np))
    return out[:N]


# speedup vs baseline: 1.7329x; 1.7329x over previous
import functools

import jax
import jax.numpy as jnp
import numpy as np
from jax import lax
from jax.experimental import pallas as pl
from jax.experimental.pallas import tpu as pltpu
from jax.experimental.pallas import tpu_sc as plsc

N = 10000
NPAD = 10240
E = 320000
CHUNK = 128
NCHUNKS = 2560
CH_PER_TILE = NCHUNKS // 32
EPAD = NCHUNKS * CHUNK
HEADS = 8
BASES = 4
F_H = 16
F_B = BASES * F_H
ROWS_PER_TILE = NPAD // 16
BLK = 256
GRID = 40

_mesh = plsc.VectorSubcoreMesh(core_axis_name="c", subcore_axis_name="s")
_sc_params = pltpu.CompilerParams(use_tc_tiling_on_sc=False)
_sc_params_nl = pltpu.CompilerParams(use_tc_tiling_on_sc=False,
                                     needs_layout_passes=False)


@functools.partial(
    pl.kernel,
    out_type=jax.ShapeDtypeStruct((2, NPAD, 16), jnp.float32),
    mesh=_mesh,
    compiler_params=_sc_params,
    scratch_types=[
        pltpu.VMEM((CH_PER_TILE, CHUNK), jnp.int32),
        pltpu.VMEM((CHUNK, 16), jnp.float32),
        pltpu.VMEM_SHARED((NPAD, 16), jnp.float32),
        pltpu.SemaphoreType.DMA,
    ],
)
def _sc_degree(col_hbm, out_hbm, col_v, ones_v, deg_sh, sem):
    cid = lax.axis_index("c")
    sid = lax.axis_index("s")
    wid = sid * 2 + cid

    z16 = jnp.zeros((16,), jnp.float32)

    @pl.loop(0, CHUNK)
    def _(i):
        ones_v[i, pl.ds(0, 16)] = z16

    @pl.loop(0, ROWS_PER_TILE // CHUNK)
    def _(k):
        pltpu.sync_copy(ones_v, deg_sh.at[pl.ds(sid * ROWS_PER_TILE + k * CHUNK, CHUNK)])

    o16 = jnp.ones((16,), jnp.float32)

    @pl.loop(0, CHUNK)
    def _(i):
        ones_v[i, pl.ds(0, 16)] = o16

    pltpu.sync_copy(col_hbm.at[pl.ds(wid * CH_PER_TILE, CH_PER_TILE)], col_v)
    plsc.subcore_barrier()

    @pl.loop(0, CH_PER_TILE)
    def _(j):
        pltpu.async_copy(ones_v, deg_sh.at[col_v.at[j]], sem, add=True)

    @pl.loop(0, CH_PER_TILE)
    def _(j):
        pltpu.make_async_copy(ones_v, deg_sh.at[col_v.at[j]], sem).wait()

    plsc.subcore_barrier()
    pltpu.sync_copy(
        deg_sh.at[pl.ds(sid * ROWS_PER_TILE, ROWS_PER_TILE)],
        out_hbm.at[cid, pl.ds(sid * ROWS_PER_TILE, ROWS_PER_TILE)],
    )


ECHUNK = 2048
F_T = F_B // 16
ECORE = EPAD // 2
NECH = ECORE // ECHUNK


@functools.partial(
    pl.kernel,
    out_type=jax.ShapeDtypeStruct((2, 16, F_T * NPAD), jnp.float32),
    mesh=_mesh,
    compiler_params=_sc_params_nl,
    scratch_types=[
        pltpu.VMEM((F_T * NPAD,), jnp.float32),
        pltpu.VMEM((F_T * NPAD,), jnp.float32),
        pltpu.VMEM((2, ECHUNK), jnp.int32),
        pltpu.VMEM((2, ECHUNK), jnp.int32),
        pltpu.SemaphoreType.DMA,
        pltpu.SemaphoreType.DMA,
        pltpu.SemaphoreType.DMA,
    ],
)
def _sc_agg(b2_hbm, row_hbm, col_hbm, out_hbm, b2_v, agg_v, row_v, col_v,
            sem_r, sem_c, sem_b):
    cid = lax.axis_index("c")
    sid = lax.axis_index("s")

    for j in range(F_T):
        pltpu.async_copy(b2_hbm.at[pl.ds(sid * F_T * N + j * N, N)],
                         b2_v.at[pl.ds(j * NPAD, N)], sem_b)

    z16 = jnp.zeros((16,), jnp.float32)

    @pl.loop(0, F_T * NPAD, step=16)
    def _(i):
        agg_v[pl.ds(i, 16)] = z16

    ebase = cid * ECORE

    def _fetch(ch, slot):
        pltpu.async_copy(
            row_hbm.at[pl.ds(ebase + ch * ECHUNK, ECHUNK)], row_v.at[slot], sem_r)
        pltpu.async_copy(
            col_hbm.at[pl.ds(ebase + ch * ECHUNK, ECHUNK)], col_v.at[slot], sem_c)

    def _wait(ch, slot):
        pltpu.make_async_copy(
            row_hbm.at[pl.ds(ebase + ch * ECHUNK, ECHUNK)], row_v.at[slot], sem_r).wait()
        pltpu.make_async_copy(
            col_hbm.at[pl.ds(ebase + ch * ECHUNK, ECHUNK)], col_v.at[slot], sem_c).wait()

    def _process(slot):
        @plsc.parallel_loop(0, ECHUNK, step=16, unroll=4)
        def _(e):
            r16 = row_v[slot, pl.ds(e, 16)]
            c16 = col_v[slot, pl.ds(e, 16)]
            for j in range(F_T):
                rf = r16 if j == 0 else r16 + (j * NPAD)
                cf = c16 if j == 0 else c16 + (j * NPAD)
                v = plsc.load_gather(b2_v, [rf])
                plsc.addupdate_scatter(agg_v, [cf], v)

    _fetch(0, 0)
    for j in range(F_T):
        pltpu.make_async_copy(b2_hbm.at[pl.ds(sid * F_T * N, N)],
                              b2_v.at[pl.ds(0, N)], sem_b).wait()

    @pl.loop(0, NECH, step=2)
    def _(ch):
        _wait(ch, 0)
        _fetch(ch + 1, 1)
        _process(0)
        _wait(ch + 1, 1)

        @pl.when(ch + 2 < NECH)
        def _():
            _fetch(ch + 2, 0)

        _process(1)

    pltpu.sync_copy(agg_v, out_hbm.at[cid, sid])


def _dense_body(x_ref, wb_ref, wc_ref, bc_ref, wr_ref, br_ref, bt_ref, wt_ref, r_ref):
    xb = x_ref[...]
    bt_ref[...] = lax.dot_general(wb_ref[...], xb, (((0,), (1,)), ((), ())),
                                  preferred_element_type=jnp.float32)
    wt_ref[...] = jnp.dot(xb, wc_ref[...], preferred_element_type=jnp.float32) + bc_ref[...]
    r_ref[...] = jnp.dot(xb, wr_ref[...], preferred_element_type=jnp.float32) + br_ref[...]


_dense = pl.pallas_call(
    _dense_body,
    grid=(GRID,),
    in_specs=[
        pl.BlockSpec((BLK, 128), lambda i: (i, 0)),
        pl.BlockSpec((128, F_B), lambda i: (0, 0)),
        pl.BlockSpec((128, HEADS * BASES), lambda i: (0, 0)),
        pl.BlockSpec((1, HEADS * BASES), lambda i: (0, 0)),
        pl.BlockSpec((128, 128), lambda i: (0, 0)),
        pl.BlockSpec((1, 128), lambda i: (0, 0)),
    ],
    out_specs=[
        pl.BlockSpec((F_B, BLK), lambda i: (0, i)),
        pl.BlockSpec((BLK, HEADS * BASES), lambda i: (i, 0)),
        pl.BlockSpec((BLK, 128), lambda i: (i, 0)),
    ],
    out_shape=[
        jax.ShapeDtypeStruct((F_B, N), jnp.float32),
        jax.ShapeDtypeStruct((N, HEADS * BASES), jnp.float32),
        jax.ShapeDtypeStruct((N, 128), jnp.float32),
    ],
)


def _scale_body(dp0_ref, dp1_ref, bases_ref, eye_ref, b2_ref, dis_ref):
    d0 = dp0_ref[...]
    d1 = dp1_ref[...]
    deg_col = d0[0, :, 0:1] + d1[0, :, 0:1] + 1.0
    dis_col = lax.rsqrt(deg_col)
    dis_row = lax.dot_general(dis_col, eye_ref[...], (((0,), (0,)), ((), ())),
                              preferred_element_type=jnp.float32)
    dis_ref[...] = dis_row
    b2_ref[...] = bases_ref[...] * dis_row


_scale = pl.pallas_call(
    _scale_body,
    grid=(GRID,),
    in_specs=[
        pl.BlockSpec((1, BLK, 16), lambda i: (0, i, 0)),
        pl.BlockSpec((1, BLK, 16), lambda i: (1, i, 0)),
        pl.BlockSpec((F_B, BLK), lambda i: (0, i)),
        pl.BlockSpec((BLK, BLK), lambda i: (0, 0)),
    ],
    out_specs=[
        pl.BlockSpec((F_B, BLK), lambda i: (0, i)),
        pl.BlockSpec((1, BLK), lambda i: (0, i)),
    ],
    out_shape=[
        jax.ShapeDtypeStruct((F_B, N), jnp.float32),
        jax.ShapeDtypeStruct((1, N), jnp.float32),
    ],
)


_P_np = np.zeros((BASES, HEADS * BASES, 128), np.float32)
_Q_np = np.zeros((BASES, F_B, 128), np.float32)
for _b in range(BASES):
    for _h in range(HEADS):
        for _f in range(F_H):
            _P_np[_b, _h * BASES + _b, _h * F_H + _f] = 1.0
            _Q_np[_b, _b * F_H + _f, _h * F_H + _f] = 1.0


def _finish_body(a0_ref, a1_ref, dis_ref, bases_ref, wt_ref, res_ref, bc_ref,
                 g_ref, bt_ref, p_ref, q_ref, o_ref):
    disr = dis_ref[...]
    a_t = a0_ref[...][0] + a1_ref[...][0]
    aggf_t = disr * a_t + (disr * disr) * bases_ref[...]
    wt = wt_ref[...]
    conv = None
    for b in range(BASES):
        we = jnp.dot(wt, p_ref[b], preferred_element_type=jnp.float32)
        ae = lax.dot_general(aggf_t, q_ref[b], (((0,), (0,)), ((), ())),
                             preferred_element_type=jnp.float32)
        t = we * ae
        conv = t if conv is None else conv + t
    o = conv + bc_ref[...] + res_ref[...]
    mu = jnp.mean(o, axis=1, keepdims=True)
    var = jnp.mean((o - mu) * (o - mu), axis=1, keepdims=True)
    o = (o - mu) * lax.rsqrt(var + 1e-5) * g_ref[...] + bt_ref[...]
    o_ref[...] = jnp.maximum(o, 0.0)


_finish = pl.pallas_call(
    _finish_body,
    grid=(GRID,),
    in_specs=[
        pl.BlockSpec((1, F_B, BLK), lambda i: (0, 0, i)),
        pl.BlockSpec((1, F_B, BLK), lambda i: (1, 0, i)),
        pl.BlockSpec((1, BLK), lambda i: (0, i)),
        pl.BlockSpec((F_B, BLK), lambda i: (0, i)),
        pl.BlockSpec((BLK, HEADS * BASES), lambda i: (i, 0)),
        pl.BlockSpec((BLK, 128), lambda i: (i, 0)),
        pl.BlockSpec((1, 128), lambda i: (0, 0)),
        pl.BlockSpec((1, 128), lambda i: (0, 0)),
        pl.BlockSpec((1, 128), lambda i: (0, 0)),
        pl.BlockSpec((BASES, HEADS * BASES, 128), lambda i: (0, 0, 0)),
        pl.BlockSpec((BASES, F_B, 128), lambda i: (0, 0, 0)),
    ],
    out_specs=pl.BlockSpec((BLK, 128), lambda i: (i, 0)),
    out_shape=jax.ShapeDtypeStruct((N, 128), jnp.float32),
)


def kernel(x, edge_index, W_bases, W_comb, b_comb, bias_conv, W_res, b_res,
           ln_gamma, ln_beta):
    row = edge_index[0]
    col = edge_index[1]
    pad = jnp.full((EPAD - E,), N, jnp.int32)
    row_p = jnp.concatenate([row, pad])
    col_p = jnp.concatenate([col, pad])

    bases_t, wt, res = _dense(x, W_bases, W_comb, b_comb.reshape(1, -1),
                              W_res, b_res.reshape(1, -1))
    degp = _sc_degree(col_p.reshape(NCHUNKS, CHUNK))
    b2_t, dis = _scale(degp, degp, bases_t, jnp.eye(BLK, dtype=jnp.float32))
    agg_t = _sc_agg(b2_t.reshape(-1), row_p, col_p)
    agg2 = agg_t.reshape(2, F_B, NPAD)
    out = _finish(agg2, agg2, dis, bases_t, wt, res,
                  bias_conv.reshape(1, -1), ln_gamma.reshape(1, -1),
                  ln_beta.reshape(1, -1), jnp.asarray(_P_np), jnp.asarray(_Q_np))
    return out


# speedup vs baseline: 2.0094x; 1.1596x over previous
import functools

import jax
import jax.numpy as jnp
import numpy as np
from jax import lax
from jax.experimental import pallas as pl
from jax.experimental.pallas import tpu as pltpu
from jax.experimental.pallas import tpu_sc as plsc

N = 10000
NPAD = 10240
E = 320000
CHUNK = 128
NCHUNKS = 2560
CH_PER_TILE = NCHUNKS // 32
EPAD = NCHUNKS * CHUNK
HEADS = 8
BASES = 4
F_H = 16
F_B = BASES * F_H
ROWS_PER_TILE = NPAD // 16
BLK = 256
GRID = 40

_mesh = plsc.VectorSubcoreMesh(core_axis_name="c", subcore_axis_name="s")
_sc_params = pltpu.CompilerParams(use_tc_tiling_on_sc=False,
                                  needs_layout_passes=False)


@functools.partial(
    pl.kernel,
    out_type=jax.ShapeDtypeStruct((2, NPAD // 128, 128), jnp.float32),
    mesh=_mesh,
    compiler_params=_sc_params,
    scratch_types=[
        pltpu.VMEM((CH_PER_TILE, CHUNK), jnp.int32),
        pltpu.VMEM((CHUNK, 16), jnp.float32),
        pltpu.VMEM((ROWS_PER_TILE, 16), jnp.float32),
        pltpu.VMEM((ROWS_PER_TILE // 128, 128), jnp.float32),
        pltpu.VMEM_SHARED((NPAD, 16), jnp.float32),
        pltpu.SemaphoreType.DMA,
    ],
)
def _sc_degree(col_hbm, out_hbm, col_v, ones_v, deg_loc, compact_v, deg_sh, sem):
    cid = lax.axis_index("c")
    sid = lax.axis_index("s")
    wid = sid * 2 + cid

    z16 = jnp.zeros((16,), jnp.float32)

    @pl.loop(0, CHUNK)
    def _(i):
        ones_v[i, pl.ds(0, 16)] = z16

    @pl.loop(0, ROWS_PER_TILE // CHUNK)
    def _(k):
        pltpu.sync_copy(ones_v, deg_sh.at[pl.ds(sid * ROWS_PER_TILE + k * CHUNK, CHUNK)])

    o16 = jnp.ones((16,), jnp.float32)

    @pl.loop(0, CHUNK)
    def _(i):
        ones_v[i, pl.ds(0, 16)] = o16

    pltpu.sync_copy(col_hbm.at[pl.ds(wid * CH_PER_TILE, CH_PER_TILE)], col_v)
    plsc.subcore_barrier()

    @pl.loop(0, CH_PER_TILE)
    def _(j):
        pltpu.async_copy(ones_v, deg_sh.at[col_v.at[j]], sem, add=True)

    @pl.loop(0, CH_PER_TILE)
    def _(j):
        pltpu.make_async_copy(ones_v, deg_sh.at[col_v.at[j]], sem).wait()

    plsc.subcore_barrier()

    pltpu.sync_copy(deg_sh.at[pl.ds(sid * ROWS_PER_TILE, ROWS_PER_TILE)], deg_loc)
    ivec16 = lax.iota(jnp.int32, 16)
    zvec16 = jnp.zeros((16,), jnp.int32)

    @pl.loop(0, ROWS_PER_TILE // 128)
    def _(gg):
        for k in range(8):
            v = plsc.load_gather(deg_loc, [ivec16 + (gg * 128 + 16 * k), zvec16])
            compact_v[gg, pl.ds(k * 16, 16)] = v

    pltpu.sync_copy(
        compact_v,
        out_hbm.at[cid, pl.ds(sid * (ROWS_PER_TILE // 128), ROWS_PER_TILE // 128)],
    )


F_T = F_B // 16
ECH_ROWS = 16
ROWS_CORE = NCHUNKS // 2
NECH = ROWS_CORE // ECH_ROWS


@functools.partial(
    pl.kernel,
    out_type=jax.ShapeDtypeStruct((2, F_B, NPAD), jnp.float32),
    mesh=_mesh,
    compiler_params=_sc_params,
    scratch_types=[
        pltpu.VMEM((F_T, NPAD), jnp.float32),
        pltpu.VMEM((F_T, NPAD), jnp.float32),
        pltpu.VMEM((2, ECH_ROWS, CHUNK), jnp.int32),
        pltpu.VMEM((2, ECH_ROWS, CHUNK), jnp.int32),
        pltpu.SemaphoreType.DMA,
        pltpu.SemaphoreType.DMA,
        pltpu.SemaphoreType.DMA,
    ],
)
def _sc_agg(b2_hbm, row_hbm, col_hbm, out_hbm, b2_v, agg_v, row_v, col_v,
            sem_r, sem_c, sem_b):
    cid = lax.axis_index("c")
    sid = lax.axis_index("s")

    b2cp = pltpu.async_copy(b2_hbm.at[pl.ds(F_T * sid, F_T)], b2_v, sem_b)

    z16 = jnp.zeros((16,), jnp.float32)
    for j in range(F_T):
        @pl.loop(0, NPAD, step=16)
        def _(i):
            agg_v[j, pl.ds(i, 16)] = z16

    rbase = cid * ROWS_CORE

    def _fetch(ch, slot):
        pltpu.async_copy(
            row_hbm.at[pl.ds(rbase + ch * ECH_ROWS, ECH_ROWS)], row_v.at[slot], sem_r)
        pltpu.async_copy(
            col_hbm.at[pl.ds(rbase + ch * ECH_ROWS, ECH_ROWS)], col_v.at[slot], sem_c)

    def _wait(ch, slot):
        pltpu.make_async_copy(
            row_hbm.at[pl.ds(rbase + ch * ECH_ROWS, ECH_ROWS)], row_v.at[slot], sem_r).wait()
        pltpu.make_async_copy(
            col_hbm.at[pl.ds(rbase + ch * ECH_ROWS, ECH_ROWS)], col_v.at[slot], sem_c).wait()

    jvecs = [jnp.full((16,), j, jnp.int32) for j in range(F_T)]

    def _process(slot):
        @plsc.parallel_loop(0, ECH_ROWS, unroll=4)
        def _(r):
            for k in range(CHUNK // 16):
                r16 = row_v[slot, r, pl.ds(k * 16, 16)]
                c16 = col_v[slot, r, pl.ds(k * 16, 16)]
                for j in range(F_T):
                    v = plsc.load_gather(b2_v, [jvecs[j], r16])
                    plsc.addupdate_scatter(agg_v, [jvecs[j], c16], v)

    _fetch(0, 0)
    b2cp.wait()

    @pl.loop(0, NECH, step=2)
    def _(ch):
        _wait(ch, 0)
        _fetch(ch + 1, 1)
        _process(0)
        _wait(ch + 1, 1)

        @pl.when(ch + 2 < NECH)
        def _():
            _fetch(ch + 2, 0)

        _process(1)

    pltpu.sync_copy(agg_v, out_hbm.at[cid, pl.ds(F_T * sid, F_T)])


def _dense_body(x_ref, wb_ref, wc_ref, bc_ref, wr_ref, br_ref, bt_ref, wt_ref, r_ref):
    xb = x_ref[...]
    bt_ref[...] = lax.dot_general(wb_ref[...], xb, (((0,), (1,)), ((), ())),
                                  preferred_element_type=jnp.float32)
    wt_ref[...] = jnp.dot(xb, wc_ref[...], preferred_element_type=jnp.float32) + bc_ref[...]
    r_ref[...] = jnp.dot(xb, wr_ref[...], preferred_element_type=jnp.float32) + br_ref[...]


_dense = pl.pallas_call(
    _dense_body,
    grid=(GRID,),
    in_specs=[
        pl.BlockSpec((BLK, 128), lambda i: (i, 0)),
        pl.BlockSpec((128, F_B), lambda i: (0, 0)),
        pl.BlockSpec((128, HEADS * BASES), lambda i: (0, 0)),
        pl.BlockSpec((1, HEADS * BASES), lambda i: (0, 0)),
        pl.BlockSpec((128, 128), lambda i: (0, 0)),
        pl.BlockSpec((1, 128), lambda i: (0, 0)),
    ],
    out_specs=[
        pl.BlockSpec((F_B, BLK), lambda i: (0, i)),
        pl.BlockSpec((BLK, HEADS * BASES), lambda i: (i, 0)),
        pl.BlockSpec((BLK, 128), lambda i: (i, 0)),
    ],
    out_shape=[
        jax.ShapeDtypeStruct((F_B, NPAD), jnp.float32),
        jax.ShapeDtypeStruct((N, HEADS * BASES), jnp.float32),
        jax.ShapeDtypeStruct((N, 128), jnp.float32),
    ],
)


SBLK = 1024


def _scale_body(dp0_ref, dp1_ref, bases_ref, b2_ref, dis_ref):
    d = dp0_ref[...][0] + dp1_ref[...][0]
    deg_row = jnp.concatenate([d[r:r + 1, :] for r in range(8)], axis=1) + 1.0
    dis_row = lax.rsqrt(deg_row)
    dis_ref[...] = dis_row
    b2_ref[...] = bases_ref[...] * dis_row


_scale = pl.pallas_call(
    _scale_body,
    grid=(NPAD // SBLK,),
    in_specs=[
        pl.BlockSpec((1, 8, 128), lambda i: (0, i, 0)),
        pl.BlockSpec((1, 8, 128), lambda i: (1, i, 0)),
        pl.BlockSpec((F_B, SBLK), lambda i: (0, i)),
    ],
    out_specs=[
        pl.BlockSpec((F_B, SBLK), lambda i: (0, i)),
        pl.BlockSpec((1, SBLK), lambda i: (0, i)),
    ],
    out_shape=[
        jax.ShapeDtypeStruct((F_B, NPAD), jnp.float32),
        jax.ShapeDtypeStruct((1, NPAD), jnp.float32),
    ],
)


_P_np = np.zeros((BASES, HEADS * BASES, 128), np.float32)
_Q_np = np.zeros((BASES, F_B, 128), np.float32)
for _b in range(BASES):
    for _h in range(HEADS):
        for _f in range(F_H):
            _P_np[_b, _h * BASES + _b, _h * F_H + _f] = 1.0
            _Q_np[_b, _b * F_H + _f, _h * F_H + _f] = 1.0


def _finish_body(a0_ref, a1_ref, dis_ref, bases_ref, wt_ref, res_ref, bc_ref,
                 g_ref, bt_ref, p_ref, q_ref, o_ref):
    disr = dis_ref[...]
    a_t = a0_ref[...][0] + a1_ref[...][0]
    aggf_t = disr * a_t + (disr * disr) * bases_ref[...]
    wt = wt_ref[...]
    conv = None
    for b in range(BASES):
        we = jnp.dot(wt, p_ref[b], preferred_element_type=jnp.float32)
        ae = lax.dot_general(aggf_t, q_ref[b], (((0,), (0,)), ((), ())),
                             preferred_element_type=jnp.float32)
        t = we * ae
        conv = t if conv is None else conv + t
    o = conv + bc_ref[...] + res_ref[...]
    mu = jnp.mean(o, axis=1, keepdims=True)
    var = jnp.mean((o - mu) * (o - mu), axis=1, keepdims=True)
    o = (o - mu) * lax.rsqrt(var + 1e-5) * g_ref[...] + bt_ref[...]
    o_ref[...] = jnp.maximum(o, 0.0)


_finish = pl.pallas_call(
    _finish_body,
    grid=(GRID,),
    in_specs=[
        pl.BlockSpec((1, F_B, BLK), lambda i: (0, 0, i)),
        pl.BlockSpec((1, F_B, BLK), lambda i: (1, 0, i)),
        pl.BlockSpec((1, BLK), lambda i: (0, i)),
        pl.BlockSpec((F_B, BLK), lambda i: (0, i)),
        pl.BlockSpec((BLK, HEADS * BASES), lambda i: (i, 0)),
        pl.BlockSpec((BLK, 128), lambda i: (i, 0)),
        pl.BlockSpec((1, 128), lambda i: (0, 0)),
        pl.BlockSpec((1, 128), lambda i: (0, 0)),
        pl.BlockSpec((1, 128), lambda i: (0, 0)),
        pl.BlockSpec((BASES, HEADS * BASES, 128), lambda i: (0, 0, 0)),
        pl.BlockSpec((BASES, F_B, 128), lambda i: (0, 0, 0)),
    ],
    out_specs=pl.BlockSpec((BLK, 128), lambda i: (i, 0)),
    out_shape=jax.ShapeDtypeStruct((N, 128), jnp.float32),
)


def kernel(x, edge_index, W_bases, W_comb, b_comb, bias_conv, W_res, b_res,
           ln_gamma, ln_beta):
    ei_p = jnp.concatenate(
        [edge_index, jnp.full((2, EPAD - E), N, jnp.int32)], axis=1)
    row_p = ei_p[0].reshape(NCHUNKS, CHUNK)
    col_p = ei_p[1].reshape(NCHUNKS, CHUNK)

    bases_t, wt, res = _dense(x, W_bases, W_comb, b_comb.reshape(1, -1),
                              W_res, b_res.reshape(1, -1))
    degp = _sc_degree(col_p)
    b2_t, dis = _scale(degp, degp, bases_t)
    aggp = _sc_agg(b2_t, row_p, col_p)
    out = _finish(aggp, aggp, dis, bases_t, wt, res,
                  bias_conv.reshape(1, -1), ln_gamma.reshape(1, -1),
                  ln_beta.reshape(1, -1), jnp.asarray(_P_np), jnp.asarray(_Q_np))
    return out


# speedup vs baseline: 2.2109x; 1.1003x over previous
import functools

import jax
import jax.numpy as jnp
import numpy as np
from jax import lax
from jax.experimental import pallas as pl
from jax.experimental.pallas import tpu as pltpu
from jax.experimental.pallas import tpu_sc as plsc

N = 10000
NPAD = 10240
E = 320000
CHUNK = 128
NCHUNKS = 2560
CH_PER_TILE = NCHUNKS // 32
EPAD = NCHUNKS * CHUNK
HEADS = 8
BASES = 4
F_H = 16
F_B = BASES * F_H
ROWS_PER_TILE = NPAD // 16
BLK = 512
GRID = 20

_mesh = plsc.VectorSubcoreMesh(core_axis_name="c", subcore_axis_name="s")
_sc_params = pltpu.CompilerParams(use_tc_tiling_on_sc=False,
                                  needs_layout_passes=False)


@functools.partial(
    pl.kernel,
    out_type=jax.ShapeDtypeStruct((2, NPAD // 128, 128), jnp.float32),
    mesh=_mesh,
    compiler_params=_sc_params,
    scratch_types=[
        pltpu.VMEM((CH_PER_TILE, CHUNK), jnp.int32),
        pltpu.VMEM((CHUNK, 16), jnp.float32),
        pltpu.VMEM((ROWS_PER_TILE, 16), jnp.float32),
        pltpu.VMEM((ROWS_PER_TILE // 128, 128), jnp.float32),
        pltpu.VMEM_SHARED((NPAD, 16), jnp.float32),
        pltpu.SemaphoreType.DMA,
    ],
)
def _sc_degree(col_hbm, out_hbm, col_v, ones_v, deg_loc, compact_v, deg_sh, sem):
    cid = lax.axis_index("c")
    sid = lax.axis_index("s")
    wid = sid * 2 + cid

    z16 = jnp.zeros((16,), jnp.float32)

    @pl.loop(0, CHUNK)
    def _(i):
        ones_v[i, pl.ds(0, 16)] = z16

    @pl.loop(0, ROWS_PER_TILE // CHUNK)
    def _(k):
        pltpu.sync_copy(ones_v, deg_sh.at[pl.ds(sid * ROWS_PER_TILE + k * CHUNK, CHUNK)])

    o16 = jnp.ones((16,), jnp.float32)

    @pl.loop(0, CHUNK)
    def _(i):
        ones_v[i, pl.ds(0, 16)] = o16

    pltpu.sync_copy(col_hbm.at[pl.ds(wid * CH_PER_TILE, CH_PER_TILE)], col_v)
    plsc.subcore_barrier()

    @pl.loop(0, CH_PER_TILE)
    def _(j):
        pltpu.async_copy(ones_v, deg_sh.at[col_v.at[j]], sem, add=True)

    @pl.loop(0, CH_PER_TILE)
    def _(j):
        pltpu.make_async_copy(ones_v, deg_sh.at[col_v.at[j]], sem).wait()

    plsc.subcore_barrier()

    pltpu.sync_copy(deg_sh.at[pl.ds(sid * ROWS_PER_TILE, ROWS_PER_TILE)], deg_loc)
    ivec16 = lax.iota(jnp.int32, 16)
    zvec16 = jnp.zeros((16,), jnp.int32)

    @pl.loop(0, ROWS_PER_TILE // 128)
    def _(gg):
        for k in range(8):
            v = plsc.load_gather(deg_loc, [ivec16 + (gg * 128 + 16 * k), zvec16])
            compact_v[gg, pl.ds(k * 16, 16)] = v

    pltpu.sync_copy(
        compact_v,
        out_hbm.at[cid, pl.ds(sid * (ROWS_PER_TILE // 128), ROWS_PER_TILE // 128)],
    )


F_T = F_B // 16
ECH_ROWS = 16
ROWS_CORE = NCHUNKS // 2
NECH = ROWS_CORE // ECH_ROWS


@functools.partial(
    pl.kernel,
    out_type=jax.ShapeDtypeStruct((2, F_B, NPAD), jnp.float32),
    mesh=_mesh,
    compiler_params=_sc_params,
    scratch_types=[
        pltpu.VMEM((F_T, NPAD), jnp.float32),
        pltpu.VMEM((F_T, NPAD), jnp.float32),
        pltpu.VMEM((2, ECH_ROWS, CHUNK), jnp.int32),
        pltpu.VMEM((2, ECH_ROWS, CHUNK), jnp.int32),
        pltpu.SemaphoreType.DMA,
        pltpu.SemaphoreType.DMA,
        pltpu.SemaphoreType.DMA,
    ],
)
def _sc_agg(b2_hbm, row_hbm, col_hbm, out_hbm, b2_v, agg_v, row_v, col_v,
            sem_r, sem_c, sem_b):
    cid = lax.axis_index("c")
    sid = lax.axis_index("s")

    b2cp = pltpu.async_copy(b2_hbm.at[pl.ds(F_T * sid, F_T)], b2_v, sem_b)

    z16 = jnp.zeros((16,), jnp.float32)
    for j in range(F_T):
        @pl.loop(0, NPAD, step=16)
        def _(i):
            agg_v[j, pl.ds(i, 16)] = z16

    rbase = cid * ROWS_CORE

    def _fetch(ch, slot):
        pltpu.async_copy(
            row_hbm.at[pl.ds(rbase + ch * ECH_ROWS, ECH_ROWS)], row_v.at[slot], sem_r)
        pltpu.async_copy(
            col_hbm.at[pl.ds(rbase + ch * ECH_ROWS, ECH_ROWS)], col_v.at[slot], sem_c)

    def _wait(ch, slot):
        pltpu.make_async_copy(
            row_hbm.at[pl.ds(rbase + ch * ECH_ROWS, ECH_ROWS)], row_v.at[slot], sem_r).wait()
        pltpu.make_async_copy(
            col_hbm.at[pl.ds(rbase + ch * ECH_ROWS, ECH_ROWS)], col_v.at[slot], sem_c).wait()

    jvecs = [jnp.full((16,), j, jnp.int32) for j in range(F_T)]

    def _process(slot):
        @plsc.parallel_loop(0, ECH_ROWS, unroll=4)
        def _(r):
            for k in range(CHUNK // 16):
                r16 = row_v[slot, r, pl.ds(k * 16, 16)]
                c16 = col_v[slot, r, pl.ds(k * 16, 16)]
                for j in range(F_T):
                    v = plsc.load_gather(b2_v, [jvecs[j], r16])
                    plsc.addupdate_scatter(agg_v, [jvecs[j], c16], v)

    _fetch(0, 0)
    b2cp.wait()

    @pl.loop(0, NECH, step=2)
    def _(ch):
        _wait(ch, 0)
        _fetch(ch + 1, 1)
        _process(0)
        _wait(ch + 1, 1)

        @pl.when(ch + 2 < NECH)
        def _():
            _fetch(ch + 2, 0)

        _process(1)

    pltpu.sync_copy(agg_v, out_hbm.at[cid, pl.ds(F_T * sid, F_T)])


def _dense_body(x_ref, wb_ref, wc_ref, bc_ref, wr_ref, br_ref, bt_ref, wt_ref, r_ref):
    xb = x_ref[...]
    bt_ref[...] = lax.dot_general(wb_ref[...], xb, (((0,), (1,)), ((), ())),
                                  preferred_element_type=jnp.float32)
    wt_ref[...] = jnp.dot(xb, wc_ref[...], preferred_element_type=jnp.float32) + bc_ref[...]
    r_ref[...] = jnp.dot(xb, wr_ref[...], preferred_element_type=jnp.float32) + br_ref[...]


_dense = pl.pallas_call(
    _dense_body,
    grid=(GRID,),
    in_specs=[
        pl.BlockSpec((BLK, 128), lambda i: (i, 0)),
        pl.BlockSpec((128, F_B), lambda i: (0, 0)),
        pl.BlockSpec((128, HEADS * BASES), lambda i: (0, 0)),
        pl.BlockSpec((1, HEADS * BASES), lambda i: (0, 0)),
        pl.BlockSpec((128, 128), lambda i: (0, 0)),
        pl.BlockSpec((1, 128), lambda i: (0, 0)),
    ],
    out_specs=[
        pl.BlockSpec((F_B, BLK), lambda i: (0, i)),
        pl.BlockSpec((BLK, HEADS * BASES), lambda i: (i, 0)),
        pl.BlockSpec((BLK, 128), lambda i: (i, 0)),
    ],
    out_shape=[
        jax.ShapeDtypeStruct((F_B, NPAD), jnp.float32),
        jax.ShapeDtypeStruct((N, HEADS * BASES), jnp.float32),
        jax.ShapeDtypeStruct((N, 128), jnp.float32),
    ],
)


SBLK = 1024


def _scale_body(dp0_ref, dp1_ref, bases_ref, b2_ref, dis_ref):
    d = dp0_ref[...][0] + dp1_ref[...][0]
    deg_row = jnp.concatenate([d[r:r + 1, :] for r in range(8)], axis=1) + 1.0
    dis_row = lax.rsqrt(deg_row)
    dis_ref[...] = dis_row
    b2_ref[...] = bases_ref[...] * dis_row


_scale = pl.pallas_call(
    _scale_body,
    grid=(NPAD // SBLK,),
    in_specs=[
        pl.BlockSpec((1, 8, 128), lambda i: (0, i, 0)),
        pl.BlockSpec((1, 8, 128), lambda i: (1, i, 0)),
        pl.BlockSpec((F_B, SBLK), lambda i: (0, i)),
    ],
    out_specs=[
        pl.BlockSpec((F_B, SBLK), lambda i: (0, i)),
        pl.BlockSpec((1, SBLK), lambda i: (0, i)),
    ],
    out_shape=[
        jax.ShapeDtypeStruct((F_B, NPAD), jnp.float32),
        jax.ShapeDtypeStruct((1, NPAD), jnp.float32),
    ],
)


_P_np = np.zeros((BASES, HEADS * BASES, 128), np.float32)
_Q_np = np.zeros((BASES, F_B, 128), np.float32)
for _b in range(BASES):
    for _h in range(HEADS):
        for _f in range(F_H):
            _P_np[_b, _h * BASES + _b, _h * F_H + _f] = 1.0
            _Q_np[_b, _b * F_H + _f, _h * F_H + _f] = 1.0


def _finish_body(a0_ref, a1_ref, dis_ref, bases_ref, wt_ref, res_ref, bc_ref,
                 g_ref, bt_ref, p_ref, q_ref, o_ref):
    disr = dis_ref[...]
    a_t = a0_ref[...][0] + a1_ref[...][0]
    aggf_t = disr * a_t + (disr * disr) * bases_ref[...]
    wt = wt_ref[...]
    conv = None
    for b in range(BASES):
        we = jnp.dot(wt, p_ref[b], preferred_element_type=jnp.float32)
        ae = lax.dot_general(aggf_t, q_ref[b], (((0,), (0,)), ((), ())),
                             preferred_element_type=jnp.float32)
        t = we * ae
        conv = t if conv is None else conv + t
    o = conv + bc_ref[...] + res_ref[...]
    mu = jnp.mean(o, axis=1, keepdims=True)
    var = jnp.mean((o - mu) * (o - mu), axis=1, keepdims=True)
    o = (o - mu) * lax.rsqrt(var + 1e-5) * g_ref[...] + bt_ref[...]
    o_ref[...] = jnp.maximum(o, 0.0)


_finish = pl.pallas_call(
    _finish_body,
    grid=(GRID,),
    in_specs=[
        pl.BlockSpec((1, F_B, BLK), lambda i: (0, 0, i)),
        pl.BlockSpec((1, F_B, BLK), lambda i: (1, 0, i)),
        pl.BlockSpec((1, BLK), lambda i: (0, i)),
        pl.BlockSpec((F_B, BLK), lambda i: (0, i)),
        pl.BlockSpec((BLK, HEADS * BASES), lambda i: (i, 0)),
        pl.BlockSpec((BLK, 128), lambda i: (i, 0)),
        pl.BlockSpec((1, 128), lambda i: (0, 0)),
        pl.BlockSpec((1, 128), lambda i: (0, 0)),
        pl.BlockSpec((1, 128), lambda i: (0, 0)),
        pl.BlockSpec((BASES, HEADS * BASES, 128), lambda i: (0, 0, 0)),
        pl.BlockSpec((BASES, F_B, 128), lambda i: (0, 0, 0)),
    ],
    out_specs=pl.BlockSpec((BLK, 128), lambda i: (i, 0)),
    out_shape=jax.ShapeDtypeStruct((N, 128), jnp.float32),
)


def kernel(x, edge_index, W_bases, W_comb, b_comb, bias_conv, W_res, b_res,
           ln_gamma, ln_beta):
    ei_p = jnp.concatenate(
        [edge_index, jnp.full((2, EPAD - E), N, jnp.int32)], axis=1)
    row_p = ei_p[0].reshape(NCHUNKS, CHUNK)
    col_p = ei_p[1].reshape(NCHUNKS, CHUNK)

    bases_t, wt, res = _dense(x, W_bases, W_comb, b_comb.reshape(1, -1),
                              W_res, b_res.reshape(1, -1))
    degp = _sc_degree(col_p)
    b2_t, dis = _scale(degp, degp, bases_t)
    aggp = _sc_agg(b2_t, row_p, col_p)
    out = _finish(aggp, aggp, dis, bases_t, wt, res,
                  bias_conv.reshape(1, -1), ln_gamma.reshape(1, -1),
                  ln_beta.reshape(1, -1), jnp.asarray(_P_np), jnp.asarray(_Q_np))
    return out
